# baseline (speedup 1.0000x reference)
import jax
import jax.numpy as jnp
from jax import lax
from jax.experimental import pallas as pl
from jax.experimental.pallas import tpu as pltpu

_N = 1_000_000
_NIDX = 200
_VREG = 1024
_FCH = 8 * _VREG
_NFILL = _N // _FCH
_TAIL = _N - _NFILL * _FCH
_SEG = 244 * _VREG
_NSEG = 4


def _fill(buf, fconst):
    def _step(i, _):
        off = pl.multiple_of(i * _FCH, _FCH)
        buf[pl.ds(off, _FCH)] = jnp.full((_FCH,), fconst, jnp.float32)
        return 0
    lax.fori_loop(0, _NFILL, _step, 0, unroll=4)
    buf[pl.ds(_NFILL * _FCH, _TAIL)] = jnp.full((_TAIL,), fconst, jnp.float32)


def _scatter(buf, idx_s, val_s):
    iota128 = lax.broadcasted_iota(jnp.int32, (128,), 0)

    def _step(j, _):
        idx = idx_s[j]
        base = pl.multiple_of((idx // 128) * 128, 128)
        lane = idx - base
        chunk = buf[pl.ds(base, 128)]
        buf[pl.ds(base, 128)] = jnp.where(iota128 == lane, val_s[j], chunk)
        return 0
    lax.fori_loop(0, _NIDX, _step, 0, unroll=8)


def _send(buf, out_h, sem):
    copies = []
    for k in range(_NSEG):
        copies.append(pltpu.async_copy(
            buf.at[pl.ds(k * _SEG, _SEG)], out_h.at[pl.ds(k * _SEG, _SEG)],
            sem))
    copies.append(pltpu.async_copy(
        buf.at[pl.ds(_NSEG * _SEG, _TAIL)],
        out_h.at[pl.ds(_NSEG * _SEG, _TAIL)], sem))
    return copies


def _tc_body(user_h, item_h, rating_h, tuser_h, titem_h, trating_h,
             ouser_h, orating_h, otuser_h, otrating_h,
             bufa, bufb, ubuf_v, tubuf_v, user_m, item_m, rating_m, tuser_m,
             titem_m, trating_m, sema, semb, semin, semu):
    incopies = [
        pltpu.async_copy(user_h, user_m, semin),
        pltpu.async_copy(item_h, item_m, semin),
        pltpu.async_copy(rating_h, rating_m, semin),
        pltpu.async_copy(tuser_h, tuser_m, semin),
        pltpu.async_copy(titem_h, titem_m, semin),
        pltpu.async_copy(trating_h, trating_m, semin),
    ]
    _fill(bufa, jnp.float32(0.0))
    for cp in incopies:
        cp.wait()
    ubuf_v[...] = jnp.full((_NIDX,), user_m[0], jnp.int32)
    tubuf_v[...] = jnp.full((_NIDX,), tuser_m[0], jnp.int32)
    cpu_u = pltpu.async_copy(ubuf_v, ouser_h, semu)
    cpu_t = pltpu.async_copy(tubuf_v, otuser_h, semu)
    _scatter(bufa, item_m, rating_m)
    cpa = _send(bufa, orating_h, sema)
    _fill(bufb, jnp.float32(jnp.nan))
    _scatter(bufb, titem_m, trating_m)
    cpb = _send(bufb, otrating_h, semb)
    for cp in cpa + cpb + [cpu_u, cpu_t]:
        cp.wait()


def kernel(user, item, rating, target_user, target_item, target_rating):
    out_shape = (
        jax.ShapeDtypeStruct((_NIDX,), jnp.int32),
        jax.ShapeDtypeStruct((_N,), jnp.float32),
        jax.ShapeDtypeStruct((_NIDX,), jnp.int32),
        jax.ShapeDtypeStruct((_N,), jnp.float32),
    )
    anyspec = pl.BlockSpec(memory_space=pl.ANY)
    return pl.pallas_call(
        _tc_body,
        in_specs=[anyspec] * 6,
        out_specs=[anyspec] * 4,
        out_shape=out_shape,
        scratch_shapes=[
            pltpu.VMEM((_N,), jnp.float32),
            pltpu.VMEM((_N,), jnp.float32),
            pltpu.VMEM((_NIDX,), jnp.int32),
            pltpu.VMEM((_NIDX,), jnp.int32),
            pltpu.SMEM((1,), jnp.int32),
            pltpu.SMEM((_NIDX,), jnp.int32),
            pltpu.SMEM((_NIDX,), jnp.float32),
            pltpu.SMEM((1,), jnp.int32),
            pltpu.SMEM((_NIDX,), jnp.int32),
            pltpu.SMEM((_NIDX,), jnp.float32),
            pltpu.SemaphoreType.DMA,
            pltpu.SemaphoreType.DMA,
            pltpu.SemaphoreType.DMA,
            pltpu.SemaphoreType.DMA,
        ],
    )(user, item, rating, target_user, target_item, target_rating)


# speedup vs baseline: 1.0392x; 1.0392x over previous
import jax
import jax.numpy as jnp
from jax import lax
from jax.experimental import pallas as pl
from jax.experimental.pallas import tpu as pltpu

_N = 1_000_000
_NIDX = 200
_VREG = 1024
_FCH = 8 * _VREG
_NFILL = _N // _FCH
_TAIL = _N - _NFILL * _FCH
_SEG = 122 * _VREG
_NSEG = 8


def _fill(buf, fconst):
    def _step(i, _):
        off = pl.multiple_of(i * _FCH, _FCH)
        buf[pl.ds(off, _FCH)] = jnp.full((_FCH,), fconst, jnp.float32)
        return 0
    lax.fori_loop(0, _NFILL, _step, 0, unroll=8)
    buf[pl.ds(_NFILL * _FCH, _TAIL)] = jnp.full((_TAIL,), fconst, jnp.float32)


def _scatter(buf, idx_s, val_s):
    iota128 = lax.broadcasted_iota(jnp.int32, (128,), 0)

    def _step(j, _):
        idx = idx_s[j]
        base = pl.multiple_of((idx // 128) * 128, 128)
        lane = idx - base
        chunk = buf[pl.ds(base, 128)]
        buf[pl.ds(base, 128)] = jnp.where(iota128 == lane, val_s[j], chunk)
        return 0
    lax.fori_loop(0, _NIDX, _step, 0, unroll=20)


def _send(buf, out_h, sem):
    copies = []
    for k in range(_NSEG):
        copies.append(pltpu.async_copy(
            buf.at[pl.ds(k * _SEG, _SEG)], out_h.at[pl.ds(k * _SEG, _SEG)],
            sem))
    copies.append(pltpu.async_copy(
        buf.at[pl.ds(_NSEG * _SEG, _TAIL)],
        out_h.at[pl.ds(_NSEG * _SEG, _TAIL)], sem))
    return copies


def _tc_body(user_h, item_h, rating_h, tuser_h, titem_h, trating_h,
             ouser_h, orating_h, otuser_h, otrating_h,
             bufa, bufb, ubuf_v, tubuf_v, user_m, item_m, rating_m, tuser_m,
             titem_m, trating_m, sema, semb, semin, semu):
    incopies = [
        pltpu.async_copy(user_h, user_m, semin),
        pltpu.async_copy(item_h, item_m, semin),
        pltpu.async_copy(rating_h, rating_m, semin),
        pltpu.async_copy(tuser_h, tuser_m, semin),
        pltpu.async_copy(titem_h, titem_m, semin),
        pltpu.async_copy(trating_h, trating_m, semin),
    ]
    _fill(bufa, jnp.float32(0.0))
    for cp in incopies:
        cp.wait()
    _scatter(bufa, item_m, rating_m)
    cpa = _send(bufa, orating_h, sema)
    ubuf_v[...] = jnp.full((_NIDX,), user_m[0], jnp.int32)
    tubuf_v[...] = jnp.full((_NIDX,), tuser_m[0], jnp.int32)
    cpu_u = pltpu.async_copy(ubuf_v, ouser_h, semu)
    cpu_t = pltpu.async_copy(tubuf_v, otuser_h, semu)
    _fill(bufb, jnp.float32(jnp.nan))
    _scatter(bufb, titem_m, trating_m)
    cpb = _send(bufb, otrating_h, semb)
    for cp in cpa + cpb + [cpu_u, cpu_t]:
        cp.wait()


def kernel(user, item, rating, target_user, target_item, target_rating):
    out_shape = (
        jax.ShapeDtypeStruct((_NIDX,), jnp.int32),
        jax.ShapeDtypeStruct((_N,), jnp.float32),
        jax.ShapeDtypeStruct((_NIDX,), jnp.int32),
        jax.ShapeDtypeStruct((_N,), jnp.float32),
    )
    anyspec = pl.BlockSpec(memory_space=pl.ANY)
    return pl.pallas_call(
        _tc_body,
        in_specs=[anyspec] * 6,
        out_specs=[anyspec] * 4,
        out_shape=out_shape,
        scratch_shapes=[
            pltpu.VMEM((_N,), jnp.float32),
            pltpu.VMEM((_N,), jnp.float32),
            pltpu.VMEM((_NIDX,), jnp.int32),
            pltpu.VMEM((_NIDX,), jnp.int32),
            pltpu.SMEM((1,), jnp.int32),
            pltpu.SMEM((_NIDX,), jnp.int32),
            pltpu.SMEM((_NIDX,), jnp.float32),
            pltpu.SMEM((1,), jnp.int32),
            pltpu.SMEM((_NIDX,), jnp.int32),
            pltpu.SMEM((_NIDX,), jnp.float32),
            pltpu.SemaphoreType.DMA,
            pltpu.SemaphoreType.DMA,
            pltpu.SemaphoreType.DMA,
            pltpu.SemaphoreType.DMA,
        ],
    )(user, item, rating, target_user, target_item, target_rating)
